# baseline (speedup 1.0000x reference)
import functools

import jax
import jax.numpy as jnp
from jax import lax
from jax.experimental import pallas as pl
from jax.experimental.pallas import tpu as pltpu
from jax.experimental.pallas import tpu_sc as plsc

V = 1000000
D = 64
R = 16
B = 16384
L = 20

NC, NS = 2, 16
NW = NC * NS
N = B * L
PW = N // NW
CH = 640
NCH = PW // CH
SUB = CH // 128
GRP = CH // L
VPG = L * D // 16
IPW = PW // 128


def _adap_body(idx0_ref, wd_ref, bd_ref, wu_ref, bu_ref, table_ref,
               out_ref, emb_ref, sem):
    for i in range(L):
        pltpu.make_async_copy(
            table_ref.at[pl.ds(idx0_ref[i], 1)], emb_ref.at[pl.ds(i, 1)], sem
        ).start()
    for i in range(L):
        pltpu.make_async_copy(
            table_ref.at[pl.ds(idx0_ref[i], 1)], emb_ref.at[pl.ds(i, 1)], sem
        ).wait()
    h = emb_ref[...]
    mid = jnp.maximum(
        jnp.dot(h, wd_ref[...], preferred_element_type=jnp.float32)
        + bd_ref[...], 0.0)
    out_ref[...] = (h
                    + jnp.dot(mid, wu_ref[...],
                              preferred_element_type=jnp.float32)
                    + bu_ref[...])


_adapter_call = pl.pallas_call(
    _adap_body,
    out_shape=jax.ShapeDtypeStruct((L, D), jnp.float32),
    in_specs=[
        pl.BlockSpec(memory_space=pltpu.SMEM),
        pl.BlockSpec(memory_space=pltpu.VMEM),
        pl.BlockSpec(memory_space=pltpu.VMEM),
        pl.BlockSpec(memory_space=pltpu.VMEM),
        pl.BlockSpec(memory_space=pltpu.VMEM),
        pl.BlockSpec(memory_space=pltpu.MemorySpace.HBM),
    ],
    out_specs=pl.BlockSpec(memory_space=pltpu.VMEM),
    scratch_shapes=[pltpu.VMEM((L, D), jnp.float32), pltpu.SemaphoreType.DMA],
)


def _sc_body(table, idxr, adap, out, idx_v, adap_v, rows_v, gs0, gs1, ss0, ss1):
    wid = lax.axis_index("s") * NC + lax.axis_index("c")
    ibase = wid * IPW
    obase = wid * PW

    pltpu.sync_copy(idxr.at[pl.ds(ibase, IPW)], idx_v)
    pltpu.sync_copy(adap, adap_v)

    gsems = (gs0, gs1)
    ssems = (ss0, ss1)

    def start_gather(c, buf):
        for j in range(SUB):
            pltpu.async_copy(
                table.at[idx_v.at[c * SUB + j]],
                rows_v.at[buf, pl.ds(j * 128, 128), :],
                gsems[buf])

    def wait_gather(buf):
        for j in range(SUB):
            pltpu.make_async_copy(
                table.at[idx_v.at[j]],
                rows_v.at[buf, pl.ds(j * 128, 128), :],
                gsems[buf]).wait()

    def add_chunk(buf):
        @pl.loop(0, GRP)
        def _(g):
            base_r = g * L
            for v in range(VPG):
                r = base_r + (v // 4)
                col = (v % 4) * 16
                rows_v[buf, r, pl.ds(col, 16)] = (
                    rows_v[buf, r, pl.ds(col, 16)] + adap_v[v, :])

    def start_store(c, buf):
        pltpu.async_copy(
            rows_v.at[buf], out.at[pl.ds(obase + c * CH, CH)], ssems[buf])

    def wait_store(buf):
        pltpu.make_async_copy(
            rows_v.at[buf], out.at[pl.ds(obase, CH)], ssems[buf]).wait()

    start_gather(0, 0)
    start_gather(1, 1)

    @pl.loop(0, NCH, step=2)
    def _(c):
        for b in range(2):
            cc = c + b
            wait_gather(b)
            add_chunk(b)
            start_store(cc, b)

            @pl.when(cc + 2 < NCH)
            def _():
                wait_store(b)
                start_gather(cc + 2, b)

    wait_store(0)
    wait_store(1)


_sc_call = functools.partial(
    pl.kernel,
    out_type=jax.ShapeDtypeStruct((N, D), jnp.float32),
    mesh=plsc.VectorSubcoreMesh(
        core_axis_name="c", subcore_axis_name="s",
        num_cores=NC, num_subcores=NS),
    scratch_types=[
        pltpu.VMEM((IPW, 128), jnp.int32),
        pltpu.VMEM((VPG, 16), jnp.float32),
        pltpu.VMEM((2, CH, D), jnp.float32),
        pltpu.SemaphoreType.DMA,
        pltpu.SemaphoreType.DMA,
        pltpu.SemaphoreType.DMA,
        pltpu.SemaphoreType.DMA,
    ],
    compiler_params=pltpu.CompilerParams(use_tc_tiling_on_sc=False),
)(_sc_body)


def kernel(indices, table, W_down, b_down, W_up, b_up):
    idx0 = indices[0]
    adap = _adapter_call(idx0, W_down, b_down.reshape(1, R),
                         W_up, b_up.reshape(1, D), table)
    out = _sc_call(table, indices.reshape(N // 128, 128),
                   adap.reshape(VPG, 16))
    return out.reshape(B, L, D)
